# baseline (device time: 16469 ns/iter reference)
import jax
import jax.numpy as jnp
from jax import lax
from jax.experimental import pallas as pl
from jax.experimental.pallas import tpu as pltpu

N_DEV = 4
N_GLOBAL = 4096
EPS = 1e-5
BLK = 128


def kernel(x, gamma):
    m, n = x.shape
    nblk = m // BLK

    def body(
        x_hbm, g_ref, out_hbm,
        xv, tv, comm_ref,
        in_sems, out_sems, send_sems, recv_sems,
    ):
        my = lax.axis_index("i")

        in_dmas = []
        for s in range(nblk):
            dma = pltpu.make_async_copy(
                x_hbm.at[pl.ds(s * BLK, BLK), :],
                xv.at[pl.ds(s * BLK, BLK), :],
                in_sems.at[s],
            )
            dma.start()
            in_dmas.append(dma)

        g = g_ref[...]

        for s in range(nblk):
            in_dmas[s].wait()
            blk = xv[pl.ds(s * BLK, BLK), :]
            comm_ref[0, :, s : s + 1] = jnp.sum(
                blk * blk, axis=1, keepdims=True
            )
            tv[pl.ds(s * BLK, BLK), :] = (blk * g).astype(tv.dtype)

        bar = pltpu.get_barrier_semaphore()
        for j in range(1, N_DEV):
            pl.semaphore_signal(
                bar, inc=1,
                device_id=((my + j) % N_DEV,),
                device_id_type=pl.DeviceIdType.MESH,
            )
        pl.semaphore_wait(bar, N_DEV - 1)

        rdmas = []
        for j in range(1, N_DEV):
            rdma = pltpu.make_async_remote_copy(
                src_ref=comm_ref.at[0],
                dst_ref=comm_ref.at[j],
                send_sem=send_sems.at[j - 1],
                recv_sem=recv_sems.at[j - 1],
                device_id=((my + j) % N_DEV,),
                device_id_type=pl.DeviceIdType.MESH,
            )
            rdma.start()
            rdmas.append(rdma)
        for rdma in rdmas:
            rdma.wait()

        total = (
            comm_ref[0] + comm_ref[1] + comm_ref[2] + comm_ref[3]
        )
        rinv = lax.rsqrt(total / N_GLOBAL + EPS)

        out_dmas = []
        for s in range(nblk):
            tv[pl.ds(s * BLK, BLK), :] = (
                tv[pl.ds(s * BLK, BLK), :] * rinv[:, s : s + 1]
            ).astype(tv.dtype)
            dma = pltpu.make_async_copy(
                tv.at[pl.ds(s * BLK, BLK), :],
                out_hbm.at[pl.ds(s * BLK, BLK), :],
                out_sems.at[s],
            )
            dma.start()
            out_dmas.append(dma)
        for dma in out_dmas:
            dma.wait()

    return pl.pallas_call(
        body,
        out_shape=jax.ShapeDtypeStruct((m, n), jnp.bfloat16),
        in_specs=[
            pl.BlockSpec(memory_space=pl.ANY),
            pl.BlockSpec(memory_space=pltpu.VMEM),
        ],
        out_specs=pl.BlockSpec(memory_space=pl.ANY),
        scratch_shapes=[
            pltpu.VMEM((m, n), jnp.float32),
            pltpu.VMEM((m, n), jnp.bfloat16),
            pltpu.VMEM((N_DEV, BLK, nblk), jnp.float32),
            pltpu.SemaphoreType.DMA((nblk,)),
            pltpu.SemaphoreType.DMA((nblk,)),
            pltpu.SemaphoreType.DMA((N_DEV - 1,)),
            pltpu.SemaphoreType.DMA((N_DEV - 1,)),
        ],
        compiler_params=pltpu.CompilerParams(collective_id=0),
    )(x, gamma.reshape(1, n))


# device time: 11943 ns/iter; 1.3790x vs baseline; 1.3790x over previous
import jax
import jax.numpy as jnp
from jax import lax
from jax.experimental import pallas as pl
from jax.experimental.pallas import tpu as pltpu

N_DEV = 4
N_GLOBAL = 4096
EPS = 1e-5
BLK = 128


def kernel(x, gamma):
    m, n = x.shape
    nblk = m // BLK

    def body(x_ref, g_ref, out_ref, tv, p128, comm_ref, send_sems, recv_sems):
        my = lax.axis_index("i")

        bar = pltpu.get_barrier_semaphore()
        for j in range(1, N_DEV):
            pl.semaphore_signal(
                bar, inc=1,
                device_id=((my + j) % N_DEV,),
                device_id_type=pl.DeviceIdType.MESH,
            )

        for s in range(nblk):
            blk = x_ref[pl.ds(s * BLK, BLK), :]
            p128[:, s : s + 1] = jnp.sum(blk * blk, axis=1, keepdims=True)

        comm_ref[0] = jnp.transpose(p128[...])[:nblk, :]

        pl.semaphore_wait(bar, N_DEV - 1)

        rdmas = []
        for j in range(1, N_DEV):
            rdma = pltpu.make_async_remote_copy(
                src_ref=comm_ref.at[0],
                dst_ref=comm_ref.at[j],
                send_sem=send_sems.at[j - 1],
                recv_sem=recv_sems.at[j - 1],
                device_id=((my + j) % N_DEV,),
                device_id_type=pl.DeviceIdType.MESH,
            )
            rdma.start()
            rdmas.append(rdma)

        g = g_ref[...]
        for s in range(nblk):
            tv[pl.ds(s * BLK, BLK), :] = (
                x_ref[pl.ds(s * BLK, BLK), :] * g
            ).astype(tv.dtype)

        for rdma in rdmas:
            rdma.wait()

        total = (
            comm_ref[0] + comm_ref[1] + comm_ref[2] + comm_ref[3]
        )
        rinv16 = lax.rsqrt(total / N_GLOBAL + EPS)
        rinv = jnp.transpose(rinv16).astype(jnp.bfloat16)
        for s in range(nblk):
            out_ref[pl.ds(s * BLK, BLK), :] = (
                tv[pl.ds(s * BLK, BLK), :] * rinv[:, s : s + 1]
            )

    return pl.pallas_call(
        body,
        out_shape=jax.ShapeDtypeStruct((m, n), jnp.bfloat16),
        in_specs=[
            pl.BlockSpec(memory_space=pltpu.VMEM),
            pl.BlockSpec(memory_space=pltpu.VMEM),
        ],
        out_specs=pl.BlockSpec(memory_space=pltpu.VMEM),
        scratch_shapes=[
            pltpu.VMEM((m, n), jnp.bfloat16),
            pltpu.VMEM((BLK, 128), jnp.float32),
            pltpu.VMEM((N_DEV, nblk, 128), jnp.float32),
            pltpu.SemaphoreType.DMA((N_DEV - 1,)),
            pltpu.SemaphoreType.DMA((N_DEV - 1,)),
        ],
        compiler_params=pltpu.CompilerParams(collective_id=0),
    )(x, gamma.reshape(1, n))
